# baseline (speedup 1.0000x reference)
import functools

import jax
import jax.numpy as jnp
from jax import lax
from jax.experimental import pallas as pl
from jax.experimental.pallas import tpu as pltpu
from jax.experimental.pallas import tpu_sc as plsc

NCLASSES_ = 100000
EMB_D = 20
BATCH_ = 1024


_NC, _NS = 2, 16
_NW = _NC * _NS
_BPW = BATCH_ // _NW
_DP = 24


def _sc_gather(table_p, idx):
    mesh = plsc.VectorSubcoreMesh(core_axis_name="c", subcore_axis_name="s")

    @functools.partial(
        pl.kernel,
        mesh=mesh,
        compiler_params=pltpu.CompilerParams(use_tc_tiling_on_sc=False),
        out_type=jax.ShapeDtypeStruct((BATCH_, _DP), jnp.float32),
        scratch_types=[
            pltpu.VMEM((_BPW,), jnp.int32),
            pltpu.VMEM((_BPW, _DP), jnp.float32),
            pltpu.SemaphoreType.DMA,
        ],
    )
    def gather_kernel(table_hbm, idx_hbm, out_hbm, idx_v, rows_v, sem):
        wid = lax.axis_index("s") * _NC + lax.axis_index("c")
        base = wid * _BPW
        pltpu.sync_copy(idx_hbm.at[pl.ds(base, _BPW)], idx_v)
        pltpu.async_copy(table_hbm.at[idx_v], rows_v, sem).wait()
        pltpu.sync_copy(rows_v, out_hbm.at[pl.ds(base, _BPW)])

    return gather_kernel(table_p, idx)


_VT = 2048


def _mm_kernel(x_ref, w_ref, b_ref, o_ref):
    acc = lax.dot_general(
        x_ref[:, :EMB_D], w_ref[...],
        (((1,), (1,)), ((), ())),
        preferred_element_type=jnp.float32,
    )
    o_ref[...] = acc + b_ref[...]


def _tc_linear(x, W, b2d):
    grid = pl.cdiv(NCLASSES_, _VT)
    return pl.pallas_call(
        _mm_kernel,
        grid=(grid,),
        in_specs=[
            pl.BlockSpec((BATCH_, _DP), lambda j: (0, 0)),
            pl.BlockSpec((_VT, EMB_D), lambda j: (j, 0)),
            pl.BlockSpec((1, _VT), lambda j: (0, j)),
        ],
        out_specs=pl.BlockSpec((BATCH_, _VT), lambda j: (0, j)),
        out_shape=jax.ShapeDtypeStruct((BATCH_, NCLASSES_), jnp.float32),
    )(x, W, b2d)


def kernel(input, emb_table, W, b):
    idx = input.astype(jnp.int32)
    table_p = jnp.pad(emb_table, ((0, 0), (0, _DP - EMB_D)))
    x = _sc_gather(table_p, idx)
    return _tc_linear(x, W, b.reshape(1, NCLASSES_))
